# baseline (speedup 1.0000x reference)
import functools

import jax
import jax.numpy as jnp
from jax import lax
from jax.experimental import pallas as pl
from jax.experimental.pallas import tpu as pltpu
from jax.experimental.pallas import tpu_sc as plsc

N = 10000
E = 320000
D = 128
HID = 64

NC = 2
NS = 16
FPT = HID // NS
EHALF = E // NC
CHUNK = 6400
NCHUNK = EHALF // CHUNK
VPC = CHUNK // 16


def _proj_body(h_ref, w1t_ref, w1b_ref, b1_ref, p_ref, q_ref):
    hb = h_ref[...]
    p_ref[...] = (
        jnp.dot(hb, w1t_ref[...], preferred_element_type=jnp.float32)
        + b1_ref[...]
    )
    q_ref[...] = jnp.dot(hb, w1b_ref[...], preferred_element_type=jnp.float32)


def _finish_body(part_ref, b2_ref, out_ref):
    z = jnp.sum(part_ref[...], axis=0) + b2_ref[0, 0]
    out_ref[...] = jax.nn.sigmoid(z)


def _sc_body(tbl_hbm, src_hbm, dst_hbm, w2_hbm, part_hbm,
             tbl, sidx, didx, part, w2v):
    c = lax.axis_index("c")
    s = lax.axis_index("s")

    pltpu.sync_copy(tbl_hbm.at[s], tbl)
    pltpu.sync_copy(w2_hbm, w2v)

    w2s = [
        plsc.load_gather(w2v, [jnp.full((16,), s * FPT + j, jnp.int32)])
        for j in range(FPT)
    ]
    w2d = [2.0 * w for w in w2s]
    ksum = w2s[0] + w2s[1] + w2s[2] + w2s[3]

    base = c * EHALF

    def chunk_body(g, _):
        off = base + g * CHUNK
        pltpu.sync_copy(src_hbm.at[pl.ds(off, CHUNK)], sidx)
        pltpu.sync_copy(dst_hbm.at[pl.ds(off, CHUNK)], didx)

        @plsc.parallel_loop(0, VPC, unroll=16)
        def vec_body(v):
            o = v * 16
            sv = sidx[pl.ds(o, 16)] * (2 * FPT)
            dv = didx[pl.ds(o, 16)] * (2 * FPT)
            acc = ksum
            for j in range(FPT):
                p = plsc.load_gather(tbl, [sv + j])
                q = plsc.load_gather(tbl, [dv + (FPT + j)])
                d = jnp.exp(p + q) + 1.0
                acc = acc - w2d[j] / d
            part[pl.ds(o, 16)] = acc
        pltpu.sync_copy(part, part_hbm.at[s, pl.ds(off, CHUNK)])
        return 0

    lax.fori_loop(0, NCHUNK, chunk_body, 0)


def kernel(h, edge_index, W1, b1, W2, b2):
    p, q = pl.pallas_call(
        _proj_body,
        out_shape=[
            jax.ShapeDtypeStruct((N, HID), jnp.float32),
            jax.ShapeDtypeStruct((N, HID), jnp.float32),
        ],
    )(h, W1[:D], W1[D:], b1.reshape(1, HID))

    pr = (2.0 * p).reshape(N, NS, FPT).transpose(1, 0, 2)
    qr = (2.0 * q).reshape(N, NS, FPT).transpose(1, 0, 2)
    tbl = jnp.concatenate([pr, qr], axis=2).reshape(NS, N * 2 * FPT)

    src = edge_index[0]
    dst = edge_index[1]
    w2 = jnp.pad(W2.reshape(HID), (0, 64))

    mesh = plsc.VectorSubcoreMesh(core_axis_name="c", subcore_axis_name="s")
    sc = pl.kernel(
        _sc_body,
        out_type=jax.ShapeDtypeStruct((NS, E), jnp.float32),
        mesh=mesh,
        compiler_params=pltpu.CompilerParams(needs_layout_passes=False),
        scratch_types=[
            pltpu.VMEM((N * 2 * FPT,), jnp.float32),
            pltpu.VMEM((CHUNK,), jnp.int32),
            pltpu.VMEM((CHUNK,), jnp.int32),
            pltpu.VMEM((CHUNK,), jnp.float32),
            pltpu.VMEM((2 * HID,), jnp.float32),
        ],
    )
    partials = sc(tbl, src, dst, w2)

    pr3 = partials.reshape(NS, E // 128, 128)
    out = pl.pallas_call(
        _finish_body,
        in_specs=[
            pl.BlockSpec(memory_space=pltpu.VMEM),
            pl.BlockSpec(memory_space=pltpu.SMEM),
        ],
        out_specs=pl.BlockSpec(memory_space=pltpu.VMEM),
        out_shape=jax.ShapeDtypeStruct((E // 128, 128), jnp.float32),
    )(pr3, b2.reshape(1, 1))

    return out.reshape(E, 1)


# speedup vs baseline: 1.0965x; 1.0965x over previous
import functools

import jax
import jax.numpy as jnp
from jax import lax
from jax.experimental import pallas as pl
from jax.experimental.pallas import tpu as pltpu
from jax.experimental.pallas import tpu_sc as plsc

N = 10000
E = 320000
D = 128
HID = 64

NC = 2
NS = 16
FPT = HID // NS
EHALF = E // NC
CHUNK = 8192
EPADH = 163840
NCHUNK = EPADH // CHUNK
VPC = CHUNK // 16


def _proj_body(h_ref, w1t_ref, w1b_ref, b1_ref, p_ref, q_ref):
    hb = h_ref[...]
    p_ref[...] = (
        jnp.dot(hb, w1t_ref[...], preferred_element_type=jnp.float32)
        + b1_ref[...]
    )
    q_ref[...] = jnp.dot(hb, w1b_ref[...], preferred_element_type=jnp.float32)


CS = CHUNK // NS


def _sc_body(tbl_hbm, src_hbm, dst_hbm, w2_hbm, out_hbm, part_hbm,
             tbl, sidx, didx, part, gath, outv, w2v):
    c = lax.axis_index("c")
    s = lax.axis_index("s")

    pltpu.sync_copy(tbl_hbm.at[s], tbl)
    pltpu.sync_copy(w2_hbm, w2v)

    w2s = [
        plsc.load_gather(w2v, [jnp.full((16,), s * FPT + j, jnp.int32)])
        for j in range(FPT)
    ]
    w2d = [2.0 * w for w in w2s]
    ksum = w2s[0] + w2s[1] + w2s[2] + w2s[3]
    b2s = plsc.load_gather(w2v, [jnp.full((16,), HID, jnp.int32)])

    base = c * EPADH

    def chunk_body(g, _):
        off = base + g * CHUNK
        pltpu.sync_copy(src_hbm.at[pl.ds(off, CHUNK)], sidx)
        pltpu.sync_copy(dst_hbm.at[pl.ds(off, CHUNK)], didx)

        @plsc.parallel_loop(0, VPC, unroll=8)
        def vec_body(v):
            o = v * 16
            sv = sidx[pl.ds(o, 16)] * (2 * FPT)
            dv = didx[pl.ds(o, 16)] * (2 * FPT)
            acc = ksum
            for j in range(FPT):
                p = plsc.load_gather(tbl, [sv + j])
                q = plsc.load_gather(tbl, [dv + (FPT + j)])
                d = jnp.exp(p + q) + 1.0
                acc = acc - w2d[j] / d
            part[pl.ds(o, 16)] = acc

        pltpu.sync_copy(part, part_hbm.at[s, pl.ds(off, CHUNK)])
        plsc.subcore_barrier()
        pltpu.sync_copy(part_hbm.at[:, pl.ds(off + s * CS, CS)], gath)

        @plsc.parallel_loop(0, CS // 16, unroll=4)
        def red_body(u):
            o = u * 16
            t = gath[0, pl.ds(o, 16)]
            for r in range(1, NS):
                t = t + gath[r, pl.ds(o, 16)]
            z = t + b2s
            outv[pl.ds(o, 16)] = 1.0 / (1.0 + jnp.exp(-z))

        pltpu.sync_copy(outv, out_hbm.at[pl.ds(off + s * CS, CS)])
        return 0

    lax.fori_loop(0, NCHUNK, chunk_body, 0)


def kernel(h, edge_index, W1, b1, W2, b2):
    p, q = pl.pallas_call(
        _proj_body,
        out_shape=[
            jax.ShapeDtypeStruct((N, HID), jnp.float32),
            jax.ShapeDtypeStruct((N, HID), jnp.float32),
        ],
    )(h, W1[:D], W1[D:], b1.reshape(1, HID))

    pr = (2.0 * p).reshape(N, NS, FPT).transpose(1, 0, 2)
    qr = (2.0 * q).reshape(N, NS, FPT).transpose(1, 0, 2)
    tbl = jnp.concatenate([pr, qr], axis=2).reshape(NS, N * 2 * FPT)

    zpad = jnp.zeros((EPADH - EHALF,), jnp.int32)
    src = jnp.concatenate(
        [edge_index[0, :EHALF], zpad, edge_index[0, EHALF:], zpad])
    dst = jnp.concatenate(
        [edge_index[1, :EHALF], zpad, edge_index[1, EHALF:], zpad])
    w2 = jnp.pad(jnp.concatenate([W2.reshape(HID), b2]), (0, 63))

    mesh = plsc.VectorSubcoreMesh(core_axis_name="c", subcore_axis_name="s")
    sc = pl.kernel(
        _sc_body,
        out_type=[
            jax.ShapeDtypeStruct((NC * EPADH,), jnp.float32),
            jax.ShapeDtypeStruct((NS, NC * EPADH), jnp.float32),
        ],
        mesh=mesh,
        compiler_params=pltpu.CompilerParams(needs_layout_passes=False),
        scratch_types=[
            pltpu.VMEM((N * 2 * FPT,), jnp.float32),
            pltpu.VMEM((CHUNK,), jnp.int32),
            pltpu.VMEM((CHUNK,), jnp.int32),
            pltpu.VMEM((CHUNK,), jnp.float32),
            pltpu.VMEM((NS, CS), jnp.float32),
            pltpu.VMEM((CS,), jnp.float32),
            pltpu.VMEM((2 * HID,), jnp.float32),
        ],
    )
    out, _ = sc(tbl, src, dst, w2)
    out = jnp.concatenate([out[:EHALF], out[EPADH:EPADH + EHALF]])
    return out.reshape(E, 1)


# speedup vs baseline: 1.1222x; 1.0234x over previous
import jax
import jax.numpy as jnp
from jax import lax
from jax.experimental import pallas as pl
from jax.experimental.pallas import tpu as pltpu
from jax.experimental.pallas import tpu_sc as plsc

N = 10000
E = 320000
D = 128
HID = 64

NC = 2
NS = 16
FPT = HID // NS
EHALF = E // NC
CHUNK = 6400
NCHUNK = EHALF // CHUNK
VPC = CHUNK // 16


def _proj_body(h_ref, w1t_ref, w1b_ref, b1_ref, p_ref, q_ref):
    hb = h_ref[...]
    p_ref[...] = (
        jnp.dot(hb, w1t_ref[...], preferred_element_type=jnp.float32)
        + b1_ref[...]
    )
    q_ref[...] = jnp.dot(hb, w1b_ref[...], preferred_element_type=jnp.float32)


def _finish_body(part_ref, b2_ref, out_ref):
    z = jnp.sum(part_ref[...], axis=0) + b2_ref[0, 0]
    out_ref[...] = jax.nn.sigmoid(z)


def _sc_body(tbl_hbm, src_hbm, dst_hbm, w2_hbm, part_hbm,
             tbl, sidx, didx, part, w2v):
    c = lax.axis_index("c")
    s = lax.axis_index("s")

    pltpu.sync_copy(tbl_hbm.at[s], tbl)
    pltpu.sync_copy(w2_hbm, w2v)

    w2s = [
        plsc.load_gather(w2v, [jnp.full((16,), s * FPT + j, jnp.int32)])
        for j in range(FPT)
    ]
    w2d = [2.0 * w for w in w2s]
    ksum = w2s[0] + w2s[1] + w2s[2] + w2s[3]

    base = c * EHALF

    def chunk_body(g, _):
        off = base + g * CHUNK
        pltpu.sync_copy(src_hbm.at[pl.ds(off, CHUNK)], sidx)
        pltpu.sync_copy(dst_hbm.at[pl.ds(off, CHUNK)], didx)

        @plsc.parallel_loop(0, VPC, unroll=8)
        def vec_body(v):
            o = v * 16
            sv = sidx[pl.ds(o, 16)]
            dv = didx[pl.ds(o, 16)]
            acc = ksum
            for j in range(FPT):
                p = plsc.load_gather(tbl, [sv + j] if j else [sv])
                q = plsc.load_gather(tbl, [dv + j] if j else [dv])
                d = jnp.exp(p + q) + 1.0
                acc = acc - w2d[j] / d
            part[pl.ds(o, 16)] = acc

        pltpu.sync_copy(part, part_hbm.at[s, pl.ds(off, CHUNK)])
        return 0

    lax.fori_loop(0, NCHUNK, chunk_body, 0)


def kernel(h, edge_index, W1, b1, W2, b2):
    p, q = pl.pallas_call(
        _proj_body,
        out_shape=[
            jax.ShapeDtypeStruct((N, HID), jnp.float32),
            jax.ShapeDtypeStruct((N, HID), jnp.float32),
        ],
    )(h, W1[:D], W1[D:], b1.reshape(1, HID))

    pr = (2.0 * p).reshape(N, NS, FPT).transpose(1, 0, 2)
    qr = (2.0 * q).reshape(N, NS, FPT).transpose(1, 0, 2)
    tbl = jnp.concatenate([pr, qr], axis=2).reshape(NS, N * 2 * FPT)

    src = edge_index[0] * (2 * FPT)
    dst = edge_index[1] * (2 * FPT) + FPT
    w2 = jnp.pad(W2.reshape(HID), (0, 64))

    mesh = plsc.VectorSubcoreMesh(core_axis_name="c", subcore_axis_name="s")
    sc = pl.kernel(
        _sc_body,
        out_type=jax.ShapeDtypeStruct((NS, E), jnp.float32),
        mesh=mesh,
        compiler_params=pltpu.CompilerParams(needs_layout_passes=False),
        scratch_types=[
            pltpu.VMEM((N * 2 * FPT,), jnp.float32),
            pltpu.VMEM((CHUNK,), jnp.int32),
            pltpu.VMEM((CHUNK,), jnp.int32),
            pltpu.VMEM((CHUNK,), jnp.float32),
            pltpu.VMEM((2 * HID,), jnp.float32),
        ],
    )
    partials = sc(tbl, src, dst, w2)

    pr3 = partials.reshape(NS, E // 128, 128)
    out = pl.pallas_call(
        _finish_body,
        in_specs=[
            pl.BlockSpec(memory_space=pltpu.VMEM),
            pl.BlockSpec(memory_space=pltpu.SMEM),
        ],
        out_specs=pl.BlockSpec(memory_space=pltpu.VMEM),
        out_shape=jax.ShapeDtypeStruct((E // 128, 128), jnp.float32),
    )(pr3, b2.reshape(1, 1))

    return out.reshape(E, 1)


# speedup vs baseline: 1.2028x; 1.0719x over previous
import jax
import jax.numpy as jnp
from jax import lax
from jax.experimental import pallas as pl
from jax.experimental.pallas import tpu as pltpu
from jax.experimental.pallas import tpu_sc as plsc

N = 10000
E = 320000
D = 128
HID = 64

NC = 2
NS = 16
FPT = HID // NS
EHALF = E // NC
CHUNK = 6400
NCHUNK = EHALF // CHUNK
VPC = CHUNK // 16


def _proj_body(h_ref, w1t_ref, w1b_ref, b1_ref, p_ref, q_ref):
    hb = h_ref[...]
    p_ref[...] = (
        jnp.dot(hb, w1t_ref[...], preferred_element_type=jnp.float32)
        + b1_ref[...]
    )
    q_ref[...] = jnp.dot(hb, w1b_ref[...], preferred_element_type=jnp.float32)


def _finish_body(part_ref, b2_ref, out_ref):
    z = jnp.sum(part_ref[...], axis=0) + b2_ref[0, 0]
    out_ref[...] = jax.nn.sigmoid(z)


def _sc_body(tbl_hbm, src_hbm, dst_hbm, w2_hbm, part_hbm,
             tbl, sidx, didx, part, w2v, sem):
    c = lax.axis_index("c")
    s = lax.axis_index("s")

    pltpu.sync_copy(tbl_hbm.at[s], tbl)
    pltpu.sync_copy(w2_hbm, w2v)

    w2s = [
        plsc.load_gather(w2v, [jnp.full((16,), s * FPT + j, jnp.int32)])
        for j in range(FPT)
    ]
    w2d = [2.0 * w for w in w2s]
    ksum = w2s[0] + w2s[1] + w2s[2] + w2s[3]

    base = c * EHALF

    def chunk_body(g, _):
        off = base + g * CHUNK
        cp1 = pltpu.async_copy(src_hbm.at[pl.ds(off, CHUNK)], sidx, sem)
        cp2 = pltpu.async_copy(dst_hbm.at[pl.ds(off, CHUNK)], didx, sem)
        cp1.wait()
        cp2.wait()

        @plsc.parallel_loop(0, VPC, unroll=8)
        def vec_body(v):
            o = v * 16
            sv = sidx[pl.ds(o, 16)] * (2 * FPT)
            dv = didx[pl.ds(o, 16)] * (2 * FPT)
            acc = ksum
            for j in range(FPT):
                p = plsc.load_gather(tbl, [sv + j])
                q = plsc.load_gather(tbl, [dv + (FPT + j)])
                d = jnp.exp(p + q) + 1.0
                acc = acc - w2d[j] / d
            part[pl.ds(o, 16)] = acc

        pltpu.sync_copy(part, part_hbm.at[s, pl.ds(off, CHUNK)])
        return 0

    lax.fori_loop(0, NCHUNK, chunk_body, 0)


def kernel(h, edge_index, W1, b1, W2, b2):
    p, q = pl.pallas_call(
        _proj_body,
        out_shape=[
            jax.ShapeDtypeStruct((N, HID), jnp.float32),
            jax.ShapeDtypeStruct((N, HID), jnp.float32),
        ],
    )(h, W1[:D], W1[D:], b1.reshape(1, HID))

    pr = (2.0 * p).reshape(N, NS, FPT).transpose(1, 0, 2)
    qr = (2.0 * q).reshape(N, NS, FPT).transpose(1, 0, 2)
    tbl = jnp.concatenate([pr, qr], axis=2).reshape(NS, N * 2 * FPT)

    src = edge_index[0]
    dst = edge_index[1]
    w2 = jnp.pad(W2.reshape(HID), (0, 64))

    mesh = plsc.VectorSubcoreMesh(core_axis_name="c", subcore_axis_name="s")
    sc = pl.kernel(
        _sc_body,
        out_type=jax.ShapeDtypeStruct((NS, E), jnp.float32),
        mesh=mesh,
        compiler_params=pltpu.CompilerParams(needs_layout_passes=False),
        scratch_types=[
            pltpu.VMEM((N * 2 * FPT,), jnp.float32),
            pltpu.VMEM((CHUNK,), jnp.int32),
            pltpu.VMEM((CHUNK,), jnp.int32),
            pltpu.VMEM((CHUNK,), jnp.float32),
            pltpu.VMEM((2 * HID,), jnp.float32),
            pltpu.SemaphoreType.DMA,
        ],
    )
    partials = sc(tbl, src, dst, w2)

    pr3 = partials.reshape(NS, E // 128, 128)
    out = pl.pallas_call(
        _finish_body,
        in_specs=[
            pl.BlockSpec(memory_space=pltpu.VMEM),
            pl.BlockSpec(memory_space=pltpu.SMEM),
        ],
        out_specs=pl.BlockSpec(memory_space=pltpu.VMEM),
        out_shape=jax.ShapeDtypeStruct((E // 128, 128), jnp.float32),
    )(pr3, b2.reshape(1, 1))

    return out.reshape(E, 1)


# speedup vs baseline: 1.3246x; 1.1012x over previous
import jax
import jax.numpy as jnp
from jax import lax
from jax.experimental import pallas as pl
from jax.experimental.pallas import tpu as pltpu
from jax.experimental.pallas import tpu_sc as plsc

N = 10000
E = 320000
D = 128
HID = 64

NC = 2
NS = 16
FPT = HID // NS
EHALF = E // NC
CHUNK = 6400
NCHUNK = EHALF // CHUNK
VPC = CHUNK // 16


def _proj_body(h_ref, w1t_ref, w1b_ref, b1_ref, p_ref, q_ref):
    hb = h_ref[...]
    p_ref[...] = (
        jnp.dot(hb, w1t_ref[...], preferred_element_type=jnp.float32)
        + b1_ref[...]
    )
    q_ref[...] = jnp.dot(hb, w1b_ref[...], preferred_element_type=jnp.float32)


def _finish_body(part_ref, b2_ref, out_ref):
    z = jnp.sum(part_ref[...], axis=0) + b2_ref[0, 0]
    out_ref[...] = jax.nn.sigmoid(z)


def _sc_body(tbl_hbm, src_hbm, dst_hbm, w2_hbm, part_hbm,
             tbl, sidx_a, didx_a, sidx_b, didx_b, part, w2v, sem_a, sem_b):
    c = lax.axis_index("c")
    s = lax.axis_index("s")

    pltpu.sync_copy(tbl_hbm.at[s], tbl)
    pltpu.sync_copy(w2_hbm, w2v)

    w2s = [
        plsc.load_gather(w2v, [jnp.full((16,), s * FPT + j, jnp.int32)])
        for j in range(FPT)
    ]
    w2d = [2.0 * w for w in w2s]
    ksum = w2s[0] + w2s[1] + w2s[2] + w2s[3]

    base = c * EHALF

    def start_fetch(off, si, di, sem):
        pltpu.async_copy(src_hbm.at[pl.ds(off, CHUNK)], si, sem)
        pltpu.async_copy(dst_hbm.at[pl.ds(off, CHUNK)], di, sem)

    def wait_fetch(off, si, di, sem):
        pltpu.make_async_copy(src_hbm.at[pl.ds(off, CHUNK)], si, sem).wait()
        pltpu.make_async_copy(dst_hbm.at[pl.ds(off, CHUNK)], di, sem).wait()

    def compute_chunk(off, si, di):
        @plsc.parallel_loop(0, VPC, unroll=8)
        def vec_body(v):
            o = v * 16
            sv = si[pl.ds(o, 16)] * (2 * FPT)
            dv = di[pl.ds(o, 16)] * (2 * FPT)
            acc = ksum
            for j in range(FPT):
                p = plsc.load_gather(tbl, [sv + j])
                q = plsc.load_gather(tbl, [dv + (FPT + j)])
                d = jnp.exp(p + q) + 1.0
                acc = acc - w2d[j] / d
            part[pl.ds(o, 16)] = acc

        pltpu.sync_copy(part, part_hbm.at[s, pl.ds(off, CHUNK)])

    start_fetch(base, sidx_a, didx_a, sem_a)

    def pair_body(i, _):
        off_a = base + (2 * i) * CHUNK
        off_b = off_a + CHUNK
        start_fetch(off_b, sidx_b, didx_b, sem_b)
        wait_fetch(off_a, sidx_a, didx_a, sem_a)
        compute_chunk(off_a, sidx_a, didx_a)
        start_fetch(off_a + 2 * CHUNK, sidx_a, didx_a, sem_a)
        wait_fetch(off_b, sidx_b, didx_b, sem_b)
        compute_chunk(off_b, sidx_b, didx_b)
        return 0

    lax.fori_loop(0, (NCHUNK - 1) // 2, pair_body, 0)

    last = base + (NCHUNK - 1) * CHUNK
    wait_fetch(last, sidx_a, didx_a, sem_a)
    compute_chunk(last, sidx_a, didx_a)


def kernel(h, edge_index, W1, b1, W2, b2):
    p, q = pl.pallas_call(
        _proj_body,
        out_shape=[
            jax.ShapeDtypeStruct((N, HID), jnp.float32),
            jax.ShapeDtypeStruct((N, HID), jnp.float32),
        ],
    )(h, W1[:D], W1[D:], b1.reshape(1, HID))

    pr = (2.0 * p).reshape(N, NS, FPT).transpose(1, 0, 2)
    qr = (2.0 * q).reshape(N, NS, FPT).transpose(1, 0, 2)
    tbl = jnp.concatenate([pr, qr], axis=2).reshape(NS, N * 2 * FPT)

    src = edge_index[0]
    dst = edge_index[1]
    w2 = jnp.pad(W2.reshape(HID), (0, 64))

    mesh = plsc.VectorSubcoreMesh(core_axis_name="c", subcore_axis_name="s")
    sc = pl.kernel(
        _sc_body,
        out_type=jax.ShapeDtypeStruct((NS, E), jnp.float32),
        mesh=mesh,
        compiler_params=pltpu.CompilerParams(needs_layout_passes=False),
        scratch_types=[
            pltpu.VMEM((N * 2 * FPT,), jnp.float32),
            pltpu.VMEM((CHUNK,), jnp.int32),
            pltpu.VMEM((CHUNK,), jnp.int32),
            pltpu.VMEM((CHUNK,), jnp.int32),
            pltpu.VMEM((CHUNK,), jnp.int32),
            pltpu.VMEM((CHUNK,), jnp.float32),
            pltpu.VMEM((2 * HID,), jnp.float32),
            pltpu.SemaphoreType.DMA,
            pltpu.SemaphoreType.DMA,
        ],
    )
    partials = sc(tbl, src, dst, w2)

    pr3 = partials.reshape(NS, E // 128, 128)
    out = pl.pallas_call(
        _finish_body,
        in_specs=[
            pl.BlockSpec(memory_space=pltpu.VMEM),
            pl.BlockSpec(memory_space=pltpu.SMEM),
        ],
        out_specs=pl.BlockSpec(memory_space=pltpu.VMEM),
        out_shape=jax.ShapeDtypeStruct((E // 128, 128), jnp.float32),
    )(pr3, b2.reshape(1, 1))

    return out.reshape(E, 1)


# speedup vs baseline: 1.3515x; 1.0204x over previous
import jax
import jax.numpy as jnp
from jax import lax
from jax.experimental import pallas as pl
from jax.experimental.pallas import tpu as pltpu
from jax.experimental.pallas import tpu_sc as plsc

N = 10000
E = 320000
D = 128
HID = 64

NC = 2
NS = 16
FPT = HID // NS
EHALF = E // NC
CHUNK = 6400
NCHUNK = EHALF // CHUNK
VPC = CHUNK // 16


def _proj_body(h_ref, w1t_ref, w1b_ref, b1_ref, p_ref, q_ref):
    hb = h_ref[...]
    p_ref[...] = (
        jnp.dot(hb, w1t_ref[...], preferred_element_type=jnp.float32)
        + b1_ref[...]
    )
    q_ref[...] = jnp.dot(hb, w1b_ref[...], preferred_element_type=jnp.float32)


def _finish_body(part_ref, b2_ref, out_ref):
    z = jnp.sum(part_ref[...], axis=0) + b2_ref[0, 0]
    out_ref[...] = jax.nn.sigmoid(z)


def _sc_body(tbl_hbm, src_hbm, dst_hbm, w2_hbm, part_hbm,
             tbl, sidx_a, didx_a, sidx_b, didx_b, part_a, part_b, w2v,
             sem_a, sem_b, psem_a, psem_b):
    c = lax.axis_index("c")
    s = lax.axis_index("s")

    pltpu.sync_copy(tbl_hbm.at[s], tbl)
    pltpu.sync_copy(w2_hbm, w2v)

    w2s = [
        plsc.load_gather(w2v, [jnp.full((16,), s * FPT + j, jnp.int32)])
        for j in range(FPT)
    ]
    w2d = [2.0 * w for w in w2s]
    ksum = w2s[0] + w2s[1] + w2s[2] + w2s[3]

    base = c * EHALF

    def start_fetch(off, si, di, sem):
        pltpu.async_copy(src_hbm.at[pl.ds(off, CHUNK)], si, sem)
        pltpu.async_copy(dst_hbm.at[pl.ds(off, CHUNK)], di, sem)

    def wait_fetch(off, si, di, sem):
        pltpu.make_async_copy(src_hbm.at[pl.ds(off, CHUNK)], si, sem).wait()
        pltpu.make_async_copy(dst_hbm.at[pl.ds(off, CHUNK)], di, sem).wait()

    def compute_chunk(off, si, di, pt, psem):
        @plsc.parallel_loop(0, VPC, unroll=8)
        def vec_body(v):
            o = v * 16
            sv = si[pl.ds(o, 16)] * (2 * FPT)
            dv = di[pl.ds(o, 16)] * (2 * FPT)
            acc = ksum
            for j in range(FPT):
                p = plsc.load_gather(tbl, [sv + j])
                q = plsc.load_gather(tbl, [dv + (FPT + j)])
                d = jnp.exp(p + q) + 1.0
                acc = acc - w2d[j] / d
            pt[pl.ds(o, 16)] = acc

        pltpu.async_copy(pt, part_hbm.at[s, pl.ds(off, CHUNK)], psem)

    def wait_store(off, pt, psem):
        pltpu.make_async_copy(
            pt, part_hbm.at[s, pl.ds(off, CHUNK)], psem).wait()

    start_fetch(base, sidx_a, didx_a, sem_a)

    def pair_body(i, _):
        off_a = base + (2 * i) * CHUNK
        off_b = off_a + CHUNK
        start_fetch(off_b, sidx_b, didx_b, sem_b)
        wait_fetch(off_a, sidx_a, didx_a, sem_a)
        @pl.when(i > 0)
        def _():
            wait_store(off_a - 2 * CHUNK, part_a, psem_a)
        compute_chunk(off_a, sidx_a, didx_a, part_a, psem_a)
        start_fetch(off_a + 2 * CHUNK, sidx_a, didx_a, sem_a)
        wait_fetch(off_b, sidx_b, didx_b, sem_b)

        @pl.when(i > 0)
        def _():
            wait_store(off_b - 2 * CHUNK, part_b, psem_b)
        compute_chunk(off_b, sidx_b, didx_b, part_b, psem_b)
        return 0

    npair = (NCHUNK - 1) // 2
    lax.fori_loop(0, npair, pair_body, 0)

    last = base + (NCHUNK - 1) * CHUNK
    wait_fetch(last, sidx_a, didx_a, sem_a)
    wait_store(last - 2 * CHUNK, part_a, psem_a)
    compute_chunk(last, sidx_a, didx_a, part_a, psem_a)
    wait_store(last - CHUNK, part_b, psem_b)
    wait_store(last, part_a, psem_a)


def kernel(h, edge_index, W1, b1, W2, b2):
    p, q = pl.pallas_call(
        _proj_body,
        out_shape=[
            jax.ShapeDtypeStruct((N, HID), jnp.float32),
            jax.ShapeDtypeStruct((N, HID), jnp.float32),
        ],
    )(h, W1[:D], W1[D:], b1.reshape(1, HID))

    pr = (2.0 * p).reshape(N, NS, FPT).transpose(1, 0, 2)
    qr = (2.0 * q).reshape(N, NS, FPT).transpose(1, 0, 2)
    tbl = jnp.concatenate([pr, qr], axis=2).reshape(NS, N * 2 * FPT)

    src = edge_index[0]
    dst = edge_index[1]
    w2 = jnp.pad(W2.reshape(HID), (0, 64))

    mesh = plsc.VectorSubcoreMesh(core_axis_name="c", subcore_axis_name="s")
    sc = pl.kernel(
        _sc_body,
        out_type=jax.ShapeDtypeStruct((NS, E), jnp.float32),
        mesh=mesh,
        compiler_params=pltpu.CompilerParams(needs_layout_passes=False),
        scratch_types=[
            pltpu.VMEM((N * 2 * FPT,), jnp.float32),
            pltpu.VMEM((CHUNK,), jnp.int32),
            pltpu.VMEM((CHUNK,), jnp.int32),
            pltpu.VMEM((CHUNK,), jnp.int32),
            pltpu.VMEM((CHUNK,), jnp.int32),
            pltpu.VMEM((CHUNK,), jnp.float32),
            pltpu.VMEM((CHUNK,), jnp.float32),
            pltpu.VMEM((2 * HID,), jnp.float32),
            pltpu.SemaphoreType.DMA,
            pltpu.SemaphoreType.DMA,
            pltpu.SemaphoreType.DMA,
            pltpu.SemaphoreType.DMA,
        ],
    )
    partials = sc(tbl, src, dst, w2)

    pr3 = partials.reshape(NS, E // 128, 128)
    out = pl.pallas_call(
        _finish_body,
        in_specs=[
            pl.BlockSpec(memory_space=pltpu.VMEM),
            pl.BlockSpec(memory_space=pltpu.SMEM),
        ],
        out_specs=pl.BlockSpec(memory_space=pltpu.VMEM),
        out_shape=jax.ShapeDtypeStruct((E // 128, 128), jnp.float32),
    )(pr3, b2.reshape(1, 1))

    return out.reshape(E, 1)
